# baseline (speedup 1.0000x reference)
import functools

import jax
import jax.numpy as jnp
from jax import lax
from jax.experimental import pallas as pl
from jax.experimental.pallas import tpu as pltpu
from jax.experimental.pallas import tpu_sc as plsc

B = 16384
P = 10
D = 32
V = 14
R = V * P
SR = D + 1
K = P * D
NC, NS = 2, 16
NW = NC * NS
BSL = B // NW
L = 16
BG = BSL // L


def _sc_gather(ew, pe, x_t):
    mesh = plsc.VectorSubcoreMesh(core_axis_name="c", subcore_axis_name="s")

    @functools.partial(
        pl.kernel,
        mesh=mesh,
        out_type=jax.ShapeDtypeStruct((K, B), jnp.float32),
        scratch_types=[
            pltpu.VMEM((V, D), jnp.float32),
            pltpu.VMEM((P, D), jnp.float32),
            pltpu.VMEM((R * SR,), jnp.float32),
            pltpu.VMEM((BSL,), jnp.int32),
            pltpu.VMEM((BSL,), jnp.int32),
            pltpu.VMEM((D, BSL), jnp.float32),
            pltpu.VMEM((D, BSL), jnp.float32),
            pltpu.SemaphoreType.DMA,
            pltpu.SemaphoreType.DMA,
            pltpu.SemaphoreType.DMA,
            pltpu.SemaphoreType.DMA,
        ],
        compiler_params=pltpu.CompilerParams(needs_layout_passes=False),
    )
    def k(ew_hbm, pe_hbm, x_hbm, out_hbm, ew_v, pe_v, tbl_v, xb0, xb1,
          rb0, rb1, sx0, sx1, so0, so1):
        wid = lax.axis_index("s") * NC + lax.axis_index("c")
        b0w = wid * BSL
        xbufs, rbufs = (xb0, xb1), (rb0, rb1)
        sxs, sos = (sx0, sx1), (so0, so1)

        def load_x(p):
            return pltpu.async_copy(
                x_hbm.at[p, pl.ds(b0w, BSL)], xbufs[p % 2], sxs[p % 2])

        x_pend = load_x(0)

        pltpu.sync_copy(ew_hbm, ew_v)
        pltpu.sync_copy(pe_hbm, pe_v)

        def build(c, _):
            v = c // P
            p = c - v * P
            for h in range(0, D, L):
                tbl_v[pl.ds(c * SR + h, L)] = (
                    ew_v[v, pl.ds(h, L)] + pe_v[p, pl.ds(h, L)])
            return 0

        lax.fori_loop(0, R, build, 0)

        out_pend = [None, None]
        for p in range(P):
            bp = p % 2
            nxt = load_x(p + 1) if p + 1 < P else None
            x_pend.wait()
            x_pend = nxt
            if out_pend[bp] is not None:
                out_pend[bp].wait()
            xbuf, rows = xbufs[bp], rbufs[bp]

            def cw_of(g):
                return xbuf[pl.ds(g * L, L)] * (10 * SR) + (p * SR)

            def load_grp(cw):
                return [plsc.load_gather(tbl_v, [cw + j]) for j in range(D)]

            def group(g, vals):
                cw = cw_of(g)
                new = []
                for j in range(D):
                    new.append(plsc.load_gather(tbl_v, [cw + j]))
                    rows[j, pl.ds((g - 1) * L, L)] = vals[j]
                return tuple(new)

            vals_last = lax.fori_loop(1, BG, group, tuple(load_grp(cw_of(0))))
            for j in range(D):
                rows[j, pl.ds((BG - 1) * L, L)] = vals_last[j]
            out_pend[bp] = pltpu.async_copy(
                rows, out_hbm.at[pl.ds(p * D, D), pl.ds(b0w, BSL)], sos[bp])
        for h in out_pend:
            if h is not None:
                h.wait()

    return k(ew, pe, x_t)


def kernel(x, embed_weight, pe):
    x_t = x.T.astype(jnp.int32)
    out2 = _sc_gather(embed_weight, pe, x_t)
    return out2.reshape(P, D, B).transpose(2, 0, 1)
